# Initial kernel scaffold; baseline (speedup 1.0000x reference)
import jax
import jax.numpy as jnp
from jax.experimental import pallas as pl


def kernel(scores, k):
    raise NotImplementedError("write your pallas kernel here")


# speedup vs baseline: 12.5982x; 12.5982x over previous
import jax
import jax.numpy as jnp
from jax.experimental import pallas as pl
from jax.experimental.pallas import tpu as pltpu

_ROWS = 64
_N = 32768
_RB = 8


def _body(k_ref, x_ref, out_ref, mask_ref):
    x = x_ref[...]
    bits = jax.lax.bitcast_convert_type(x, jnp.int32) & jnp.int32(0x7FFFFFFF)
    k = k_ref[0]

    def step(i, prefix):
        b = 30 - i
        cand = prefix | (jnp.int32(1) << b)
        cnt = jnp.sum((bits >= cand).astype(jnp.int32), axis=1, keepdims=True)
        return jnp.where(cnt >= k, cand, prefix)

    prefix = jax.lax.fori_loop(0, 31, step, jnp.zeros((_RB, 1), jnp.int32))
    m = bits >= prefix
    out_ref[...] = jnp.where(m, x, 0.0)
    mask_ref[...] = m.astype(jnp.int32)


def kernel(scores, k):
    karr = jnp.asarray(k, jnp.int32).reshape(1)
    out, mask = pl.pallas_call(
        _body,
        grid=(_ROWS // _RB,),
        in_specs=[
            pl.BlockSpec(memory_space=pltpu.SMEM),
            pl.BlockSpec((_RB, _N), lambda i: (i, 0)),
        ],
        out_specs=[
            pl.BlockSpec((_RB, _N), lambda i: (i, 0)),
            pl.BlockSpec((_RB, _N), lambda i: (i, 0)),
        ],
        out_shape=[
            jax.ShapeDtypeStruct((_ROWS, _N), jnp.float32),
            jax.ShapeDtypeStruct((_ROWS, _N), jnp.int32),
        ],
    )(karr, scores)
    return out, mask.astype(jnp.bool_)
